# baseline (speedup 1.0000x reference)
import jax
import jax.numpy as jnp
from jax import lax
from jax.experimental import pallas as pl
from jax.experimental.pallas import tpu as pltpu
from jax.experimental.pallas import tpu_sc as plsc

N = 10000
NPAD = 10240
E = 320000
D = 128
NC = 2
NS = 16
NW = NC * NS
CH = 64
EPW = 10240
EP = EPW * NW
NCH = EPW // CH
G = 32
NG = NCH // G
RPT = NPAD // NS


NCHAIN = 4


def _hop_body(y_hbm, eidx_hbm, zrow_hbm, out_hbm,
              idx0, idx1, b0, b1, b2, b3, acc,
              si0, si1, sg0, sg1, sg2, sg3, ss0, ss1, ss2, ss3):
    c = lax.axis_index("c")
    s = lax.axis_index("s")
    wid = c * NS + s
    base = s * RPT

    bufs = (b0, b1, b2, b3)
    sgs = (sg0, sg1, sg2, sg3)
    sss = (ss0, ss1, ss2, ss3)
    idxs = (idx0, idx1)
    sis = (si0, si1)

    def idx_group(g):
        return eidx_hbm.at[wid, pl.ds(g * G, G)], idxs[g % 2], sis[g % 2]

    src, dst, sem = idx_group(0)
    pltpu.async_copy(src, dst, sem)

    @pl.when(c == 0)
    def _():
        pltpu.sync_copy(y_hbm.at[pl.ds(base, RPT)], acc.at[pl.ds(base, RPT)])

    @pl.when(c != 0)
    def _():
        pltpu.sync_copy(zrow_hbm, acc.at[pl.ds(base, RPT)])

    plsc.subcore_barrier()

    def ridx(j):
        return idxs[(j // G) % 2].at[j % G, 0]

    def cidx(j):
        return idxs[(j // G) % 2].at[j % G, 1]

    gdesc = [None] * NCHAIN
    sdesc = [None] * NCHAIN

    src, dst, sem = idx_group(0)
    pltpu.make_async_copy(src, dst, sem).wait()
    gdesc[0] = pltpu.async_copy(y_hbm.at[ridx(0)], bufs[0], sgs[0])
    gdesc[1] = pltpu.async_copy(y_hbm.at[ridx(1)], bufs[1], sgs[1])

    for j in range(NCH):
        t = j % NCHAIN
        gdesc[t].wait()
        sdesc[t] = pltpu.async_copy(bufs[t], acc.at[cidx(j)], sss[t], add=True)
        jn = j + 2
        if jn < NCH:
            tn = jn % NCHAIN
            if sdesc[tn] is not None:
                sdesc[tn].wait()
            if jn % G == 0:
                src, dst, sem = idx_group(jn // G)
                pltpu.make_async_copy(src, dst, sem).wait()
            gdesc[tn] = pltpu.async_copy(y_hbm.at[ridx(jn)], bufs[tn], sgs[tn])
        if j % G == 1 and j // G + 1 < NG:
            src, dst, sem = idx_group(j // G + 1)
            pltpu.async_copy(src, dst, sem)

    for t in range(NCHAIN):
        if sdesc[t] is not None:
            sdesc[t].wait()

    plsc.subcore_barrier()
    pltpu.sync_copy(acc.at[pl.ds(base, RPT)], out_hbm.at[c, pl.ds(base, RPT)])


def _hop(y, eidx, zrow):
    mesh = plsc.VectorSubcoreMesh(core_axis_name="c", subcore_axis_name="s")
    return pl.kernel(
        _hop_body,
        out_type=jax.ShapeDtypeStruct((NC, NPAD, D), jnp.float32),
        mesh=mesh,
        scratch_types=[
            pltpu.VMEM((G, 2, CH), jnp.int32),
            pltpu.VMEM((G, 2, CH), jnp.int32),
            pltpu.VMEM((CH, D), jnp.float32),
            pltpu.VMEM((CH, D), jnp.float32),
            pltpu.VMEM((CH, D), jnp.float32),
            pltpu.VMEM((CH, D), jnp.float32),
            pltpu.VMEM_SHARED((NPAD, D), jnp.float32),
            pltpu.SemaphoreType.DMA,
            pltpu.SemaphoreType.DMA,
            pltpu.SemaphoreType.DMA,
            pltpu.SemaphoreType.DMA,
            pltpu.SemaphoreType.DMA,
            pltpu.SemaphoreType.DMA,
            pltpu.SemaphoreType.DMA,
            pltpu.SemaphoreType.DMA,
            pltpu.SemaphoreType.DMA,
            pltpu.SemaphoreType.DMA,
        ],
    )(y, eidx, zrow)


def _deg_body(col_hbm, zvec_hbm, ones_hbm, out_hbm, colv, onesv, acc, sem):
    c = lax.axis_index("c")
    s = lax.axis_index("s")
    wid = c * NS + s
    pltpu.sync_copy(col_hbm.at[wid], colv)
    pltpu.sync_copy(ones_hbm, onesv)
    base = s * RPT
    pltpu.sync_copy(zvec_hbm, acc.at[pl.ds(base, RPT)])
    plsc.subcore_barrier()

    def body(j, carry):
        pltpu.sync_copy(onesv, acc.at[colv.at[j]], add=True)
        return carry

    lax.fori_loop(0, NCH, body, 0)
    plsc.subcore_barrier()
    pltpu.sync_copy(acc.at[pl.ds(base, RPT)], out_hbm.at[c, pl.ds(base, RPT)])


def _deg(colr, zvec, ones):
    mesh = plsc.VectorSubcoreMesh(core_axis_name="c", subcore_axis_name="s")
    return pl.kernel(
        _deg_body,
        out_type=jax.ShapeDtypeStruct((NC, NPAD), jnp.float32),
        mesh=mesh,
        scratch_types=[
            pltpu.VMEM((NCH, CH), jnp.int32),
            pltpu.VMEM((CH,), jnp.float32),
            pltpu.VMEM_SHARED((NPAD,), jnp.float32),
            pltpu.SemaphoreType.DMA,
        ],
    )(colr, zvec, ones)


BR = 1280


def _pre_body(x_ref, d0_ref, d1_ref, o_ref):
    deg = d0_ref[...] + d1_ref[...] + 1.0
    o_ref[...] = x_ref[...] * lax.rsqrt(deg)


def _mid_body(t_ref, d0_ref, d1_ref, o_ref):
    deg = d0_ref[...] + d1_ref[...] + 1.0
    o_ref[...] = (t_ref[0] + t_ref[1]) / deg


def _fin_body(t_ref, d0_ref, d1_ref, w_ref, b_ref, o_ref):
    deg = d0_ref[...] + d1_ref[...] + 1.0
    h = (t_ref[0] + t_ref[1]) * lax.rsqrt(deg)
    o_ref[...] = lax.dot_general(
        h, w_ref[...], (((1,), (1,)), ((), ())),
        preferred_element_type=jnp.float32,
        precision=lax.Precision.HIGHEST,
    ) + b_ref[...]


def _row_spec():
    return pl.BlockSpec((BR, D), lambda i: (i, 0))


def _deg_spec():
    return pl.BlockSpec((BR, 1), lambda i: (i, 0))


def _pre(xp, d0, d1):
    return pl.pallas_call(
        _pre_body,
        grid=(NPAD // BR,),
        in_specs=[_row_spec(), _deg_spec(), _deg_spec()],
        out_specs=_row_spec(),
        out_shape=jax.ShapeDtypeStruct((NPAD, D), jnp.float32),
    )(xp, d0, d1)


def _mid(t, d0, d1):
    return pl.pallas_call(
        _mid_body,
        grid=(NPAD // BR,),
        in_specs=[pl.BlockSpec((NC, BR, D), lambda i: (0, i, 0)),
                  _deg_spec(), _deg_spec()],
        out_specs=_row_spec(),
        out_shape=jax.ShapeDtypeStruct((NPAD, D), jnp.float32),
    )(t, d0, d1)


BRF = 2000


def _fin(t, d0, d1, w, b2):
    return pl.pallas_call(
        _fin_body,
        grid=(N // BRF,),
        in_specs=[
            pl.BlockSpec((NC, BRF, D), lambda i: (0, i, 0)),
            pl.BlockSpec((BRF, 1), lambda i: (i, 0)),
            pl.BlockSpec((BRF, 1), lambda i: (i, 0)),
            pl.BlockSpec((D, D), lambda i: (0, 0)),
            pl.BlockSpec((1, D), lambda i: (0, 0)),
        ],
        out_specs=pl.BlockSpec((BRF, D), lambda i: (i, 0)),
        out_shape=jax.ShapeDtypeStruct((N, D), jnp.float32),
    )(t, d0, d1, w, b2)


def kernel(x, edge_index, W, b):
    pad_n = EP - E
    pad_row = ((jnp.arange(pad_n, dtype=jnp.int32) * 37) % N)
    pad_col = N + (jnp.arange(pad_n, dtype=jnp.int32) % (NPAD - N))
    rowr = jnp.concatenate([edge_index[0], pad_row]).reshape(NW, NCH, CH)
    colr = jnp.concatenate([edge_index[1], pad_col]).reshape(NW, NCH, CH)
    eidx = jnp.stack([rowr, colr], axis=2)
    xp = jnp.pad(x, ((0, NPAD - N), (0, 0)))
    zrow = jnp.zeros((RPT, D), jnp.float32)
    zvec = jnp.zeros((RPT,), jnp.float32)
    ones = jnp.ones((CH,), jnp.float32)

    dparts = _deg(colr, zvec, ones)
    d0 = dparts[0][:, None]
    d1 = dparts[1][:, None]
    y0 = _pre(xp, d0, d1)
    t1 = _hop(y0, eidx, zrow)
    y1 = _mid(t1, d0, d1)
    t2 = _hop(y1, eidx, zrow)
    return _fin(t2, d0, d1, W, b.reshape(1, D))


# speedup vs baseline: 1.1673x; 1.1673x over previous
import jax
import jax.numpy as jnp
from jax import lax
from jax.experimental import pallas as pl
from jax.experimental.pallas import tpu as pltpu
from jax.experimental.pallas import tpu_sc as plsc

N = 10000
NPAD = 10240
E = 320000
D = 128
NC = 2
NS = 16
NW = NC * NS
CH = 96
EPW = 10080
EP = EPW * NW
NCH = EPW // CH
G = 21
NG = NCH // G
RPT = NPAD // NS


NCHAIN = 3


def _hop_body(y_hbm, eidx_hbm, zrow_hbm, out_hbm,
              idx0, idx1, b0, b1, b2, acc,
              si0, si1, sg0, sg1, sg2, ss0, ss1, ss2):
    c = lax.axis_index("c")
    s = lax.axis_index("s")
    wid = c * NS + s
    base = s * RPT

    bufs = (b0, b1, b2)
    sgs = (sg0, sg1, sg2)
    sss = (ss0, ss1, ss2)
    idxs = (idx0, idx1)
    sis = (si0, si1)

    def idx_group(g):
        return eidx_hbm.at[wid, pl.ds(g * G, G)], idxs[g % 2], sis[g % 2]

    src, dst, sem = idx_group(0)
    pltpu.async_copy(src, dst, sem)

    @pl.when(c == 0)
    def _():
        pltpu.sync_copy(y_hbm.at[pl.ds(base, RPT)], acc.at[pl.ds(base, RPT)])

    @pl.when(c != 0)
    def _():
        pltpu.sync_copy(zrow_hbm, acc.at[pl.ds(base, RPT)])

    def ridx(j):
        return idxs[(j // G) % 2].at[j % G, 0]

    def cidx(j):
        return idxs[(j // G) % 2].at[j % G, 1]

    gdesc = [None] * NCHAIN
    sdesc = [None] * NCHAIN

    src, dst, sem = idx_group(0)
    pltpu.make_async_copy(src, dst, sem).wait()
    gdesc[0] = pltpu.async_copy(y_hbm.at[ridx(0)], bufs[0], sgs[0])
    gdesc[1] = pltpu.async_copy(y_hbm.at[ridx(1)], bufs[1], sgs[1])

    plsc.subcore_barrier()

    for j in range(NCH):
        t = j % NCHAIN
        gdesc[t].wait()
        sdesc[t] = pltpu.async_copy(bufs[t], acc.at[cidx(j)], sss[t], add=True)
        jn = j + 2
        if jn < NCH:
            tn = jn % NCHAIN
            if sdesc[tn] is not None:
                sdesc[tn].wait()
            if jn % G == 0:
                src, dst, sem = idx_group(jn // G)
                pltpu.make_async_copy(src, dst, sem).wait()
            gdesc[tn] = pltpu.async_copy(y_hbm.at[ridx(jn)], bufs[tn], sgs[tn])
        if j % G == 1 and j // G + 1 < NG:
            src, dst, sem = idx_group(j // G + 1)
            pltpu.async_copy(src, dst, sem)

    for t in range(NCHAIN):
        if sdesc[t] is not None:
            sdesc[t].wait()

    plsc.subcore_barrier()
    pltpu.sync_copy(acc.at[pl.ds(base, RPT)], out_hbm.at[c, pl.ds(base, RPT)])


def _hop(y, eidx, zrow):
    mesh = plsc.VectorSubcoreMesh(core_axis_name="c", subcore_axis_name="s")
    return pl.kernel(
        _hop_body,
        out_type=jax.ShapeDtypeStruct((NC, NPAD, D), jnp.float32),
        mesh=mesh,
        scratch_types=[
            pltpu.VMEM((G, 2, CH), jnp.int32),
            pltpu.VMEM((G, 2, CH), jnp.int32),
            pltpu.VMEM((CH, D), jnp.float32),
            pltpu.VMEM((CH, D), jnp.float32),
            pltpu.VMEM((CH, D), jnp.float32),
            pltpu.VMEM_SHARED((NPAD, D), jnp.float32),
            pltpu.SemaphoreType.DMA,
            pltpu.SemaphoreType.DMA,
            pltpu.SemaphoreType.DMA,
            pltpu.SemaphoreType.DMA,
            pltpu.SemaphoreType.DMA,
            pltpu.SemaphoreType.DMA,
            pltpu.SemaphoreType.DMA,
            pltpu.SemaphoreType.DMA,
        ],
    )(y, eidx, zrow)


def _deg_body(col_hbm, zvec_hbm, ones_hbm, out_hbm, colv, onesv, acc, sem):
    c = lax.axis_index("c")
    s = lax.axis_index("s")
    wid = c * NS + s
    pltpu.sync_copy(col_hbm.at[wid], colv)
    pltpu.sync_copy(ones_hbm, onesv)
    base = s * RPT
    pltpu.sync_copy(zvec_hbm, acc.at[pl.ds(base, RPT)])
    plsc.subcore_barrier()

    DB = 8
    prev = []
    for j0 in range(0, NCH, DB):
        batch = [
            pltpu.async_copy(onesv, acc.at[colv.at[j]], sem, add=True)
            for j in range(j0, min(j0 + DB, NCH))
        ]
        for dsc in prev:
            dsc.wait()
        prev = batch
    for dsc in prev:
        dsc.wait()
    plsc.subcore_barrier()
    pltpu.sync_copy(acc.at[pl.ds(base, RPT)], out_hbm.at[c, pl.ds(base, RPT)])


def _deg(colr, zvec, ones):
    mesh = plsc.VectorSubcoreMesh(core_axis_name="c", subcore_axis_name="s")
    return pl.kernel(
        _deg_body,
        out_type=jax.ShapeDtypeStruct((NC, NPAD), jnp.float32),
        mesh=mesh,
        scratch_types=[
            pltpu.VMEM((NCH, CH), jnp.int32),
            pltpu.VMEM((CH,), jnp.float32),
            pltpu.VMEM_SHARED((NPAD,), jnp.float32),
            pltpu.SemaphoreType.DMA,
        ],
    )(colr, zvec, ones)


BR = 1280


def _pre_body(x_ref, d0_ref, d1_ref, o_ref):
    deg = d0_ref[...] + d1_ref[...] + 1.0
    o_ref[...] = x_ref[...] * lax.rsqrt(deg)


def _mid_body(t_ref, d0_ref, d1_ref, o_ref):
    deg = d0_ref[...] + d1_ref[...] + 1.0
    o_ref[...] = (t_ref[0] + t_ref[1]) / deg


def _fin_body(t_ref, d0_ref, d1_ref, w_ref, b_ref, o_ref):
    deg = d0_ref[...] + d1_ref[...] + 1.0
    h = (t_ref[0] + t_ref[1]) * lax.rsqrt(deg)
    o_ref[...] = lax.dot_general(
        h, w_ref[...], (((1,), (1,)), ((), ())),
        preferred_element_type=jnp.float32,
        precision=lax.Precision.HIGHEST,
    ) + b_ref[...]


def _row_spec():
    return pl.BlockSpec((BR, D), lambda i: (i, 0))


def _deg_spec():
    return pl.BlockSpec((BR, 1), lambda i: (i, 0))


def _pre(xp, d0, d1):
    return pl.pallas_call(
        _pre_body,
        grid=(NPAD // BR,),
        in_specs=[_row_spec(), _deg_spec(), _deg_spec()],
        out_specs=_row_spec(),
        out_shape=jax.ShapeDtypeStruct((NPAD, D), jnp.float32),
    )(xp, d0, d1)


def _mid(t, d0, d1):
    return pl.pallas_call(
        _mid_body,
        grid=(NPAD // BR,),
        in_specs=[pl.BlockSpec((NC, BR, D), lambda i: (0, i, 0)),
                  _deg_spec(), _deg_spec()],
        out_specs=_row_spec(),
        out_shape=jax.ShapeDtypeStruct((NPAD, D), jnp.float32),
    )(t, d0, d1)


BRF = 2000


def _fin(t, d0, d1, w, b2):
    return pl.pallas_call(
        _fin_body,
        grid=(N // BRF,),
        in_specs=[
            pl.BlockSpec((NC, BRF, D), lambda i: (0, i, 0)),
            pl.BlockSpec((BRF, 1), lambda i: (i, 0)),
            pl.BlockSpec((BRF, 1), lambda i: (i, 0)),
            pl.BlockSpec((D, D), lambda i: (0, 0)),
            pl.BlockSpec((1, D), lambda i: (0, 0)),
        ],
        out_specs=pl.BlockSpec((BRF, D), lambda i: (i, 0)),
        out_shape=jax.ShapeDtypeStruct((N, D), jnp.float32),
    )(t, d0, d1, w, b2)


def kernel(x, edge_index, W, b):
    pad_n = EP - E
    pad_row = ((jnp.arange(pad_n, dtype=jnp.int32) * 37) % N)
    pad_col = N + (jnp.arange(pad_n, dtype=jnp.int32) % (NPAD - N))
    rowr = jnp.concatenate([edge_index[0], pad_row]).reshape(NW, NCH, CH)
    colr = jnp.concatenate([edge_index[1], pad_col]).reshape(NW, NCH, CH)
    eidx = jnp.stack([rowr, colr], axis=2)
    xp = jnp.pad(x, ((0, NPAD - N), (0, 0)))
    zrow = jnp.zeros((RPT, D), jnp.float32)
    zvec = jnp.zeros((RPT,), jnp.float32)
    ones = jnp.ones((CH,), jnp.float32)

    dparts = _deg(colr, zvec, ones)
    d0 = dparts[0][:, None]
    d1 = dparts[1][:, None]
    y0 = _pre(xp, d0, d1)
    t1 = _hop(y0, eidx, zrow)
    y1 = _mid(t1, d0, d1)
    t2 = _hop(y1, eidx, zrow)
    return _fin(t2, d0, d1, W, b.reshape(1, D))


# speedup vs baseline: 1.1803x; 1.0111x over previous
import jax
import jax.numpy as jnp
from jax import lax
from jax.experimental import pallas as pl
from jax.experimental.pallas import tpu as pltpu
from jax.experimental.pallas import tpu_sc as plsc

N = 10000
NPAD = 10240
E = 320000
D = 128
NC = 2
NS = 16
NW = NC * NS
CH = 112
EPW = 10080
EP = EPW * NW
NCH = EPW // CH
G = 9
NG = NCH // G
RPT = NPAD // NS


NCHAIN = 3


def _hop_body(y_hbm, eidx_hbm, zrow_hbm, out_hbm,
              idx0, idx1, b0, b1, b2, acc,
              si0, si1, sg0, sg1, sg2, ss0, ss1, ss2):
    c = lax.axis_index("c")
    s = lax.axis_index("s")
    wid = c * NS + s
    base = s * RPT

    bufs = (b0, b1, b2)
    sgs = (sg0, sg1, sg2)
    sss = (ss0, ss1, ss2)
    idxs = (idx0, idx1)
    sis = (si0, si1)

    def idx_group(g):
        return eidx_hbm.at[wid, pl.ds(g * G, G)], idxs[g % 2], sis[g % 2]

    src, dst, sem = idx_group(0)
    pltpu.async_copy(src, dst, sem)

    @pl.when(c == 0)
    def _():
        pltpu.sync_copy(y_hbm.at[pl.ds(base, RPT)], acc.at[pl.ds(base, RPT)])

    @pl.when(c != 0)
    def _():
        pltpu.sync_copy(zrow_hbm, acc.at[pl.ds(base, RPT)])

    def ridx(j):
        return idxs[(j // G) % 2].at[j % G, 0]

    def cidx(j):
        return idxs[(j // G) % 2].at[j % G, 1]

    gdesc = [None] * NCHAIN
    sdesc = [None] * NCHAIN

    src, dst, sem = idx_group(0)
    pltpu.make_async_copy(src, dst, sem).wait()
    gdesc[0] = pltpu.async_copy(y_hbm.at[ridx(0)], bufs[0], sgs[0])
    gdesc[1] = pltpu.async_copy(y_hbm.at[ridx(1)], bufs[1], sgs[1])

    plsc.subcore_barrier()

    for j in range(NCH):
        t = j % NCHAIN
        gdesc[t].wait()
        sdesc[t] = pltpu.async_copy(bufs[t], acc.at[cidx(j)], sss[t], add=True)
        jn = j + 2
        if jn < NCH:
            tn = jn % NCHAIN
            if sdesc[tn] is not None:
                sdesc[tn].wait()
            if jn % G == 0:
                src, dst, sem = idx_group(jn // G)
                pltpu.make_async_copy(src, dst, sem).wait()
            gdesc[tn] = pltpu.async_copy(y_hbm.at[ridx(jn)], bufs[tn], sgs[tn])
        if j % G == 1 and j // G + 1 < NG:
            src, dst, sem = idx_group(j // G + 1)
            pltpu.async_copy(src, dst, sem)

    for t in range(NCHAIN):
        if sdesc[t] is not None:
            sdesc[t].wait()

    plsc.subcore_barrier()
    pltpu.sync_copy(acc.at[pl.ds(base, RPT)], out_hbm.at[c, pl.ds(base, RPT)])


def _hop(y, eidx, zrow):
    mesh = plsc.VectorSubcoreMesh(core_axis_name="c", subcore_axis_name="s")
    return pl.kernel(
        _hop_body,
        out_type=jax.ShapeDtypeStruct((NC, NPAD, D), jnp.float32),
        mesh=mesh,
        scratch_types=[
            pltpu.VMEM((G, 2, CH), jnp.int32),
            pltpu.VMEM((G, 2, CH), jnp.int32),
            pltpu.VMEM((CH, D), jnp.float32),
            pltpu.VMEM((CH, D), jnp.float32),
            pltpu.VMEM((CH, D), jnp.float32),
            pltpu.VMEM_SHARED((NPAD, D), jnp.float32),
            pltpu.SemaphoreType.DMA,
            pltpu.SemaphoreType.DMA,
            pltpu.SemaphoreType.DMA,
            pltpu.SemaphoreType.DMA,
            pltpu.SemaphoreType.DMA,
            pltpu.SemaphoreType.DMA,
            pltpu.SemaphoreType.DMA,
            pltpu.SemaphoreType.DMA,
        ],
    )(y, eidx, zrow)


def _deg_body(col_hbm, zvec_hbm, ones_hbm, out_hbm, colv, onesv, acc, sem):
    c = lax.axis_index("c")
    s = lax.axis_index("s")
    wid = c * NS + s
    pltpu.sync_copy(col_hbm.at[wid], colv)
    pltpu.sync_copy(ones_hbm, onesv)
    base = s * RPT
    pltpu.sync_copy(zvec_hbm, acc.at[pl.ds(base, RPT)])
    plsc.subcore_barrier()

    DB = 8
    prev = []
    for j0 in range(0, NCH, DB):
        batch = [
            pltpu.async_copy(onesv, acc.at[colv.at[j]], sem, add=True)
            for j in range(j0, min(j0 + DB, NCH))
        ]
        for dsc in prev:
            dsc.wait()
        prev = batch
    for dsc in prev:
        dsc.wait()
    plsc.subcore_barrier()
    pltpu.sync_copy(acc.at[pl.ds(base, RPT)], out_hbm.at[c, pl.ds(base, RPT)])


def _deg(colr, zvec, ones):
    mesh = plsc.VectorSubcoreMesh(core_axis_name="c", subcore_axis_name="s")
    return pl.kernel(
        _deg_body,
        out_type=jax.ShapeDtypeStruct((NC, NPAD), jnp.float32),
        mesh=mesh,
        scratch_types=[
            pltpu.VMEM((NCH, CH), jnp.int32),
            pltpu.VMEM((CH,), jnp.float32),
            pltpu.VMEM_SHARED((NPAD,), jnp.float32),
            pltpu.SemaphoreType.DMA,
        ],
    )(colr, zvec, ones)


BR = 1280


def _pre_body(x_ref, d0_ref, d1_ref, o_ref):
    deg = d0_ref[...] + d1_ref[...] + 1.0
    o_ref[...] = x_ref[...] * lax.rsqrt(deg)


def _mid_body(t_ref, d0_ref, d1_ref, o_ref):
    deg = d0_ref[...] + d1_ref[...] + 1.0
    o_ref[...] = (t_ref[0] + t_ref[1]) / deg


def _fin_body(t_ref, d0_ref, d1_ref, w_ref, b_ref, o_ref):
    deg = d0_ref[...] + d1_ref[...] + 1.0
    h = (t_ref[0] + t_ref[1]) * lax.rsqrt(deg)
    o_ref[...] = lax.dot_general(
        h, w_ref[...], (((1,), (1,)), ((), ())),
        preferred_element_type=jnp.float32,
        precision=lax.Precision.HIGHEST,
    ) + b_ref[...]


def _row_spec():
    return pl.BlockSpec((BR, D), lambda i: (i, 0))


def _deg_spec():
    return pl.BlockSpec((BR, 1), lambda i: (i, 0))


def _pre(xp, d0, d1):
    return pl.pallas_call(
        _pre_body,
        grid=(NPAD // BR,),
        in_specs=[_row_spec(), _deg_spec(), _deg_spec()],
        out_specs=_row_spec(),
        out_shape=jax.ShapeDtypeStruct((NPAD, D), jnp.float32),
    )(xp, d0, d1)


def _mid(t, d0, d1):
    return pl.pallas_call(
        _mid_body,
        grid=(NPAD // BR,),
        in_specs=[pl.BlockSpec((NC, BR, D), lambda i: (0, i, 0)),
                  _deg_spec(), _deg_spec()],
        out_specs=_row_spec(),
        out_shape=jax.ShapeDtypeStruct((NPAD, D), jnp.float32),
    )(t, d0, d1)


BRF = 2000


def _fin(t, d0, d1, w, b2):
    return pl.pallas_call(
        _fin_body,
        grid=(N // BRF,),
        in_specs=[
            pl.BlockSpec((NC, BRF, D), lambda i: (0, i, 0)),
            pl.BlockSpec((BRF, 1), lambda i: (i, 0)),
            pl.BlockSpec((BRF, 1), lambda i: (i, 0)),
            pl.BlockSpec((D, D), lambda i: (0, 0)),
            pl.BlockSpec((1, D), lambda i: (0, 0)),
        ],
        out_specs=pl.BlockSpec((BRF, D), lambda i: (i, 0)),
        out_shape=jax.ShapeDtypeStruct((N, D), jnp.float32),
    )(t, d0, d1, w, b2)


def kernel(x, edge_index, W, b):
    pad_n = EP - E
    pad_row = ((jnp.arange(pad_n, dtype=jnp.int32) * 37) % N)
    pad_col = N + (jnp.arange(pad_n, dtype=jnp.int32) % (NPAD - N))
    rowr = jnp.concatenate([edge_index[0], pad_row]).reshape(NW, NCH, CH)
    colr = jnp.concatenate([edge_index[1], pad_col]).reshape(NW, NCH, CH)
    eidx = jnp.stack([rowr, colr], axis=2)
    xp = jnp.pad(x, ((0, NPAD - N), (0, 0)))
    zrow = jnp.zeros((RPT, D), jnp.float32)
    zvec = jnp.zeros((RPT,), jnp.float32)
    ones = jnp.ones((CH,), jnp.float32)

    dparts = _deg(colr, zvec, ones)
    d0 = dparts[0][:, None]
    d1 = dparts[1][:, None]
    y0 = _pre(xp, d0, d1)
    t1 = _hop(y0, eidx, zrow)
    y1 = _mid(t1, d0, d1)
    t2 = _hop(y1, eidx, zrow)
    return _fin(t2, d0, d1, W, b.reshape(1, D))
